# Initial kernel scaffold; baseline (speedup 1.0000x reference)
import jax
import jax.numpy as jnp
from jax.experimental import pallas as pl


def kernel(queries, keys, values, Wq, bq, Wk, bk, Wv, bv, Wo, bo):
    raise NotImplementedError("write your pallas kernel here")


# speedup vs baseline: 1.3028x; 1.3028x over previous
import math

import jax
import jax.numpy as jnp
from jax.experimental import pallas as pl
from jax.experimental.pallas import tpu as pltpu

B = 4
L = 8192
D = 1024
H = 16
DK = 64
U = 50
UP = 64
TL = 512
NL = L // TL
TLO = 2048
NLO = L // TLO
SCALE = 1.0 / math.sqrt(DK)
NEG = -1e30


def _bdot(a, b, dims):
    return jax.lax.dot_general(a.astype(jnp.bfloat16), b.astype(jnp.bfloat16),
                               dims, preferred_element_type=jnp.float32)


_MM = (((1,), (0,)), ((), ()))
_MT = (((1,), (1,)), ((), ()))


def _fused_proj_kernel(keys_ref, values_ref, queries_ref, wkt_ref, wvt_ref,
                       wqt_ref, ksbd_ref, bk_ref, bv_ref, bq_ref, mneg_ref,
                       k_ref, v_ref, vsum_ref, m_ref):
    lt = pl.program_id(1)
    k_ref[:] = _bdot(keys_ref[:], wkt_ref[:], _MM) + bk_ref[:]
    vt = _bdot(values_ref[:], wvt_ref[:], _MM) + bv_ref[:]
    v_ref[:] = vt

    @pl.when(lt == 0)
    def _():
        vsum_ref[:] = jnp.zeros_like(vsum_ref)

    vsum_ref[:] += jnp.sum(vt, axis=0, keepdims=True)

    qt = _bdot(queries_ref[:], wqt_ref[:], _MM) + bq_ref[:]
    s = _bdot(qt, ksbd_ref[:], _MM)
    sm = s + mneg_ref[:]
    cols = []
    for h in range(H):
        sh = s[:, h * UP:(h + 1) * UP]
        smh = sm[:, h * UP:(h + 1) * UP]
        mh = (jnp.max(smh, axis=1, keepdims=True)
              - jnp.sum(sh, axis=1, keepdims=True) / float(U))
        cols.append(mh)
    m_ref[:] = jnp.concatenate(cols, axis=1)


def _topk_kernel(m_ref, idx_ref):
    col = jax.lax.broadcasted_iota(jnp.int32, (H, L), 1)
    jcol = jax.lax.broadcasted_iota(jnp.int32, (H, 128), 1)

    def body(j, carry):
        m, acc = carry
        mx = jnp.max(m, axis=1, keepdims=True)
        am = jnp.min(jnp.where(m >= mx, col, L), axis=1, keepdims=True)
        acc = jnp.where(jcol == j, am, acc)
        m = jnp.where(col == am, NEG, m)
        return m, acc

    acc0 = jnp.zeros((H, 128), jnp.int32)
    _, acc = jax.lax.fori_loop(0, U, body, (m_ref[:], acc0))
    idx_ref[:] = acc


def _attn_kernel(qs_ref, k_ref, v_ref, attn_ref, upd_ref):
    s = _bdot(qs_ref[:], k_ref[:], _MT) * SCALE
    mx = jnp.max(s, axis=1, keepdims=True)
    e = jnp.exp(s - mx)
    a = e / jnp.sum(e, axis=1, keepdims=True)
    attn_ref[0] = a[:U, :]
    attn_ref[1] = a[UP:UP + U, :]
    u = _bdot(a, v_ref[:], _MM)
    upd_ref[0] = u[:UP, :DK]
    upd_ref[1] = u[UP:, DK:]


def _output_kernel(idxs_ref, perm_ref, starts_ref, base_ref, delta_ref,
                   out_ref):
    b = pl.program_id(0)
    t = pl.program_id(1)
    out_ref[:] = jnp.broadcast_to(base_ref[:], (TLO, D))
    s = starts_ref[b * (NLO + 1) + t]
    e = starts_ref[b * (NLO + 1) + t + 1]

    def body(i, carry):
        r = idxs_ref[b * (H * U) + i] - t * TLO
        p = perm_ref[b * (H * U) + i]
        out_ref[pl.ds(r, 1), :] += delta_ref[pl.ds(p, 1), :]
        return carry

    jax.lax.fori_loop(s, e, body, 0)


@jax.jit
def kernel(queries, keys, values, Wq, bq, Wk, bk, Wv, bv, Wo, bo):
    f32 = jnp.float32
    sample_idx = jax.random.randint(jax.random.key(42), (U,), 0, L)

    ksamp = keys[:, sample_idx, :] @ Wk.T + bk
    ksp = jnp.zeros((B, UP, H, DK), f32).at[:, :U].set(
        ksamp.reshape(B, U, H, DK))
    ksbd = jnp.einsum('bjhd,hg->bhdgj', ksp,
                      jnp.eye(H, dtype=f32)).reshape(B, D, H * UP)
    lane = jnp.arange(H * UP) % UP
    mneg = jnp.where(lane < U, 0.0, NEG).astype(f32).reshape(1, H * UP)

    k_arr, v_arr, vsum, m_arr = pl.pallas_call(
        _fused_proj_kernel,
        grid=(B, NL),
        in_specs=[
            pl.BlockSpec((None, TL, D), lambda b, lt: (b, lt, 0)),
            pl.BlockSpec((None, TL, D), lambda b, lt: (b, lt, 0)),
            pl.BlockSpec((None, TL, D), lambda b, lt: (b, lt, 0)),
            pl.BlockSpec((D, D), lambda b, lt: (0, 0)),
            pl.BlockSpec((D, D), lambda b, lt: (0, 0)),
            pl.BlockSpec((D, D), lambda b, lt: (0, 0)),
            pl.BlockSpec((None, D, H * UP), lambda b, lt: (b, 0, 0)),
            pl.BlockSpec((1, D), lambda b, lt: (0, 0)),
            pl.BlockSpec((1, D), lambda b, lt: (0, 0)),
            pl.BlockSpec((1, D), lambda b, lt: (0, 0)),
            pl.BlockSpec((1, H * UP), lambda b, lt: (0, 0)),
        ],
        out_specs=[
            pl.BlockSpec((None, TL, D), lambda b, lt: (b, lt, 0)),
            pl.BlockSpec((None, TL, D), lambda b, lt: (b, lt, 0)),
            pl.BlockSpec((None, 1, D), lambda b, lt: (b, 0, 0)),
            pl.BlockSpec((None, TL, H), lambda b, lt: (b, lt, 0)),
        ],
        out_shape=[
            jax.ShapeDtypeStruct((B, L, D), f32),
            jax.ShapeDtypeStruct((B, L, D), f32),
            jax.ShapeDtypeStruct((B, 1, D), f32),
            jax.ShapeDtypeStruct((B, L, H), f32),
        ],
    )(keys, values, queries, Wk.T, Wv.T, Wq.T, ksbd,
      bk.reshape(1, D), bv.reshape(1, D), bq.reshape(1, D), mneg)

    vmean = vsum.reshape(B, D) / L
    m_t = m_arr.transpose(0, 2, 1)

    idx = pl.pallas_call(
        _topk_kernel,
        grid=(B,),
        in_specs=[pl.BlockSpec((None, H, L), lambda b: (b, 0, 0))],
        out_specs=pl.BlockSpec((None, H, 128), lambda b: (b, 0, 0)),
        out_shape=jax.ShapeDtypeStruct((B, H, 128), jnp.int32),
    )(m_t)
    m_top = idx[:, :, :U]

    wq_h = Wq.reshape(H, DK, D)
    q_rows = jnp.take_along_axis(queries[:, None, :, :],
                                 m_top[..., None], axis=2)
    q_sel = (jnp.einsum('bhjm,hdm->bhjd', q_rows, wq_h)
             + bq.reshape(H, DK)[None, :, None, :])
    qp = jnp.zeros((B, H // 2, 2, UP, 2, DK), f32)
    qp = qp.at[:, :, 0, :U, 0, :].set(q_sel[:, 0::2])
    qp = qp.at[:, :, 1, :U, 1, :].set(q_sel[:, 1::2])
    qp = qp.reshape(B, H // 2, 2 * UP, 2 * DK)

    attn, upd = pl.pallas_call(
        _attn_kernel,
        grid=(B, H // 2),
        in_specs=[
            pl.BlockSpec((None, None, 2 * UP, 2 * DK), lambda b, h: (b, h, 0, 0)),
            pl.BlockSpec((None, L, 2 * DK), lambda b, h: (b, 0, h)),
            pl.BlockSpec((None, L, 2 * DK), lambda b, h: (b, 0, h)),
        ],
        out_specs=[
            pl.BlockSpec((None, 2, U, L), lambda b, h: (b, h, 0, 0)),
            pl.BlockSpec((None, 2, UP, DK), lambda b, h: (b, h, 0, 0)),
        ],
        out_shape=[
            jax.ShapeDtypeStruct((B, H, U, L), f32),
            jax.ShapeDtypeStruct((B, H, UP, DK), f32),
        ],
    )(qp, k_arr, v_arr)

    wo_h = Wo.T.reshape(H, DK, D)
    delta = jnp.einsum('bhjd,hdm->bhjm',
                       upd[:, :, :U, :] - vmean.reshape(B, H, 1, DK),
                       wo_h)
    base = vmean @ Wo.T + bo

    idx_flat = m_top.reshape(B, H * U)
    order = jnp.argsort(idx_flat, axis=1).astype(jnp.int32)
    idx_sorted = jnp.take_along_axis(idx_flat, order, axis=1)
    starts = jax.vmap(
        lambda a: jnp.searchsorted(a, jnp.arange(NLO + 1) * TLO,
                                   side='left'))(idx_sorted).astype(jnp.int32)

    output = pl.pallas_call(
        _output_kernel,
        grid_spec=pltpu.PrefetchScalarGridSpec(
            num_scalar_prefetch=3,
            grid=(B, NLO),
            in_specs=[
                pl.BlockSpec((None, 1, D), lambda b, t, *_: (b, 0, 0)),
                pl.BlockSpec((None, H * U, D), lambda b, t, *_: (b, 0, 0)),
            ],
            out_specs=pl.BlockSpec((None, TLO, D),
                                   lambda b, t, *_: (b, t, 0)),
        ),
        out_shape=jax.ShapeDtypeStruct((B, L, D), f32),
    )(idx_sorted.reshape(B * H * U), order.reshape(B * H * U),
      starts.reshape(B * (NLO + 1)), base.reshape(B, 1, D),
      delta.reshape(B, H * U, D))

    return (output, attn)


# speedup vs baseline: 1.7195x; 1.3198x over previous
import math

import jax
import jax.numpy as jnp
from jax.experimental import pallas as pl
from jax.experimental.pallas import tpu as pltpu

B = 4
L = 8192
D = 1024
H = 16
DK = 64
U = 50
UP = 64
TL = 512
NL = L // TL
TLO = 2048
NLO = L // TLO
SCALE = 1.0 / math.sqrt(DK)
NEG = -1e30


def _bdot(a, b, dims):
    return jax.lax.dot_general(a.astype(jnp.bfloat16), b.astype(jnp.bfloat16),
                               dims, preferred_element_type=jnp.float32)


_MM = (((1,), (0,)), ((), ()))
_MT = (((1,), (1,)), ((), ()))


def _fused_proj_kernel(keys_ref, values_ref, queries_ref, wkt_ref, wvt_ref,
                       wqt_ref, ksh_ref, bk_ref, bv_ref, bq_ref,
                       k_ref, v_ref, vsum_ref, m_ref):
    lt = pl.program_id(1)
    k_ref[:] = _bdot(keys_ref[:], wkt_ref[:], _MM) + bk_ref[:]
    vt = _bdot(values_ref[:], wvt_ref[:], _MM) + bv_ref[:]
    v_ref[:] = vt

    @pl.when(lt == 0)
    def _():
        vsum_ref[:] = jnp.zeros_like(vsum_ref)

    vsum_ref[:] += jnp.sum(vt, axis=0, keepdims=True)

    qt = _bdot(queries_ref[:], wqt_ref[:], _MM) + bq_ref[:]
    pad = jax.lax.broadcasted_iota(jnp.int32, (1, UP), 1) >= U
    cols = []
    for h in range(H):
        sh = _bdot(qt[:, h * DK:(h + 1) * DK], ksh_ref[h], _MM)
        smh = jnp.where(pad, NEG, sh)
        mh = (jnp.max(smh, axis=1, keepdims=True)
              - jnp.sum(sh, axis=1, keepdims=True) / float(U))
        cols.append(mh)
    m_ref[:] = jnp.concatenate(cols, axis=1)


def _topk_kernel(m_ref, idx_ref):
    col = jax.lax.broadcasted_iota(jnp.int32, (H, L), 1)
    jcol = jax.lax.broadcasted_iota(jnp.int32, (H, 128), 1)

    def body(j, carry):
        m, acc = carry
        mx = jnp.max(m, axis=1, keepdims=True)
        am = jnp.min(jnp.where(m >= mx, col, L), axis=1, keepdims=True)
        acc = jnp.where(jcol == j, am, acc)
        m = jnp.where(col == am, NEG, m)
        return m, acc

    acc0 = jnp.zeros((H, 128), jnp.int32)
    _, acc = jax.lax.fori_loop(0, U, body, (m_ref[:], acc0))
    idx_ref[:] = acc


def _attn_kernel(qs_ref, k_ref, v_ref, attn_ref, upd_ref):
    s = _bdot(qs_ref[:], k_ref[:], _MT) * SCALE
    mx = jnp.max(s, axis=1, keepdims=True)
    e = jnp.exp(s - mx)
    a = e / jnp.sum(e, axis=1, keepdims=True)
    attn_ref[0] = a[:U, :]
    attn_ref[1] = a[UP:UP + U, :]
    u = _bdot(a, v_ref[:], _MM)
    upd_ref[0] = u[:UP, :DK]
    upd_ref[1] = u[UP:, DK:]


def _output_kernel(idxs_ref, perm_ref, starts_ref, base_ref, delta_ref,
                   out_ref):
    b = pl.program_id(0)
    t = pl.program_id(1)
    out_ref[:] = jnp.broadcast_to(base_ref[:], (TLO, D))
    s = starts_ref[b * (NLO + 1) + t]
    e = starts_ref[b * (NLO + 1) + t + 1]

    def body(i, carry):
        r = idxs_ref[b * (H * U) + i] - t * TLO
        p = perm_ref[b * (H * U) + i]
        out_ref[pl.ds(r, 1), :] += delta_ref[pl.ds(p, 1), :]
        return carry

    jax.lax.fori_loop(s, e, body, 0)


@jax.jit
def kernel(queries, keys, values, Wq, bq, Wk, bk, Wv, bv, Wo, bo):
    f32 = jnp.float32
    sample_idx = jax.random.randint(jax.random.key(42), (U,), 0, L)

    ksamp = keys[:, sample_idx, :] @ Wk.T + bk
    ksp = jnp.zeros((B, UP, H, DK), f32).at[:, :U].set(
        ksamp.reshape(B, U, H, DK))
    ksh = ksp.transpose(0, 2, 3, 1)

    k_arr, v_arr, vsum, m_arr = pl.pallas_call(
        _fused_proj_kernel,
        grid=(B, NL),
        in_specs=[
            pl.BlockSpec((None, TL, D), lambda b, lt: (b, lt, 0)),
            pl.BlockSpec((None, TL, D), lambda b, lt: (b, lt, 0)),
            pl.BlockSpec((None, TL, D), lambda b, lt: (b, lt, 0)),
            pl.BlockSpec((D, D), lambda b, lt: (0, 0)),
            pl.BlockSpec((D, D), lambda b, lt: (0, 0)),
            pl.BlockSpec((D, D), lambda b, lt: (0, 0)),
            pl.BlockSpec((None, H, DK, UP), lambda b, lt: (b, 0, 0, 0)),
            pl.BlockSpec((1, D), lambda b, lt: (0, 0)),
            pl.BlockSpec((1, D), lambda b, lt: (0, 0)),
            pl.BlockSpec((1, D), lambda b, lt: (0, 0)),
        ],
        out_specs=[
            pl.BlockSpec((None, TL, D), lambda b, lt: (b, lt, 0)),
            pl.BlockSpec((None, TL, D), lambda b, lt: (b, lt, 0)),
            pl.BlockSpec((None, 1, D), lambda b, lt: (b, 0, 0)),
            pl.BlockSpec((None, TL, H), lambda b, lt: (b, lt, 0)),
        ],
        out_shape=[
            jax.ShapeDtypeStruct((B, L, D), f32),
            jax.ShapeDtypeStruct((B, L, D), f32),
            jax.ShapeDtypeStruct((B, 1, D), f32),
            jax.ShapeDtypeStruct((B, L, H), f32),
        ],
    )(keys, values, queries, Wk.T, Wv.T, Wq.T, ksh,
      bk.reshape(1, D), bv.reshape(1, D), bq.reshape(1, D))

    vmean = vsum.reshape(B, D) / L
    m_t = m_arr.transpose(0, 2, 1)

    idx = pl.pallas_call(
        _topk_kernel,
        grid=(B,),
        in_specs=[pl.BlockSpec((None, H, L), lambda b: (b, 0, 0))],
        out_specs=pl.BlockSpec((None, H, 128), lambda b: (b, 0, 0)),
        out_shape=jax.ShapeDtypeStruct((B, H, 128), jnp.int32),
    )(m_t)
    m_top = idx[:, :, :U]

    wq_h = Wq.reshape(H, DK, D)
    q_rows = jnp.take_along_axis(queries[:, None, :, :],
                                 m_top[..., None], axis=2)
    q_sel = (jnp.einsum('bhjm,hdm->bhjd', q_rows, wq_h)
             + bq.reshape(H, DK)[None, :, None, :])
    qp = jnp.zeros((B, H // 2, 2, UP, 2, DK), f32)
    qp = qp.at[:, :, 0, :U, 0, :].set(q_sel[:, 0::2])
    qp = qp.at[:, :, 1, :U, 1, :].set(q_sel[:, 1::2])
    qp = qp.reshape(B, H // 2, 2 * UP, 2 * DK)

    attn, upd = pl.pallas_call(
        _attn_kernel,
        grid=(B, H // 2),
        in_specs=[
            pl.BlockSpec((None, None, 2 * UP, 2 * DK), lambda b, h: (b, h, 0, 0)),
            pl.BlockSpec((None, L, 2 * DK), lambda b, h: (b, 0, h)),
            pl.BlockSpec((None, L, 2 * DK), lambda b, h: (b, 0, h)),
        ],
        out_specs=[
            pl.BlockSpec((None, 2, U, L), lambda b, h: (b, h, 0, 0)),
            pl.BlockSpec((None, 2, UP, DK), lambda b, h: (b, h, 0, 0)),
        ],
        out_shape=[
            jax.ShapeDtypeStruct((B, H, U, L), f32),
            jax.ShapeDtypeStruct((B, H, UP, DK), f32),
        ],
    )(qp, k_arr, v_arr)

    wo_h = Wo.T.reshape(H, DK, D)
    delta = jnp.einsum('bhjd,hdm->bhjm',
                       upd[:, :, :U, :] - vmean.reshape(B, H, 1, DK),
                       wo_h)
    base = vmean @ Wo.T + bo

    idx_flat = m_top.reshape(B, H * U)
    order = jnp.argsort(idx_flat, axis=1).astype(jnp.int32)
    idx_sorted = jnp.take_along_axis(idx_flat, order, axis=1)
    starts = jax.vmap(
        lambda a: jnp.searchsorted(a, jnp.arange(NLO + 1) * TLO,
                                   side='left'))(idx_sorted).astype(jnp.int32)

    output = pl.pallas_call(
        _output_kernel,
        grid_spec=pltpu.PrefetchScalarGridSpec(
            num_scalar_prefetch=3,
            grid=(B, NLO),
            in_specs=[
                pl.BlockSpec((None, 1, D), lambda b, t, *_: (b, 0, 0)),
                pl.BlockSpec((None, H * U, D), lambda b, t, *_: (b, 0, 0)),
            ],
            out_specs=pl.BlockSpec((None, TLO, D),
                                   lambda b, t, *_: (b, t, 0)),
        ),
        out_shape=jax.ShapeDtypeStruct((B, L, D), f32),
    )(idx_sorted.reshape(B * H * U), order.reshape(B * H * U),
      starts.reshape(B * (NLO + 1)), base.reshape(B, 1, D),
      delta.reshape(B, H * U, D))

    return (output, attn)
